# baseline (speedup 1.0000x reference)
import jax
import jax.numpy as jnp
from jax import lax
from jax.experimental import pallas as pl
from jax.experimental.pallas import tpu as pltpu
from jax.experimental.pallas import tpu_sc as plsc

_BATCH = 16384
_NFIELD = 26
_FIELD_DIM = 38462
_NROWS = _BATCH * _NFIELD
_EMB = 16

_NC = 2
_NS = 16
_NW = _NC * _NS
_B_PER_W = _NROWS // _NW
_CHUNK = 3328
_N_CHUNKS = _B_PER_W // _CHUNK
_VECS = _CHUNK // 16


def _sc_body(x_hbm, table_hbm, out_hbm, idx_v, rows_v, sem):
    wid = lax.axis_index("s") * _NC + lax.axis_index("c")
    base = wid * _B_PER_W

    def chunk_body(c, carry):
        off = pl.multiple_of(base + c * _CHUNK, _CHUNK)
        pltpu.sync_copy(x_hbm.at[pl.ds(off, _CHUNK)], idx_v)

        def add_body(i, carry2):
            sl = pl.ds(i * 16, 16)
            pos = i * 16 + lax.iota(jnp.int32, 16)
            fld = lax.rem(pos, _NFIELD)
            idx_v[sl] = idx_v[sl] + fld * _FIELD_DIM
            return carry2

        lax.fori_loop(0, _VECS, add_body, 0)
        pltpu.async_copy(table_hbm.at[idx_v], rows_v, sem).wait()
        pltpu.sync_copy(rows_v, out_hbm.at[pl.ds(off, _CHUNK)])
        return carry

    lax.fori_loop(0, _N_CHUNKS, chunk_body, 0)


def kernel(x, table):
    x_flat = x.astype(jnp.int32).reshape(_NROWS)
    run = pl.kernel(
        _sc_body,
        out_type=jax.ShapeDtypeStruct((_NROWS, _EMB), jnp.float32),
        mesh=plsc.VectorSubcoreMesh(core_axis_name="c", subcore_axis_name="s"),
        compiler_params=pltpu.CompilerParams(use_tc_tiling_on_sc=False),
        scratch_types=[
            pltpu.VMEM((_CHUNK,), jnp.int32),
            pltpu.VMEM((_CHUNK, _EMB), jnp.float32),
            pltpu.SemaphoreType.DMA,
        ],
    )
    out = run(x_flat, table)
    return out.reshape(_BATCH, _NFIELD, _EMB)
